# baseline (speedup 1.0000x reference)
import jax
import jax.numpy as jnp
from jax.experimental import pallas as pl
from jax.experimental.pallas import tpu as pltpu

_B, _H, _W = 32, 56, 56
_NP, _E, _O = 4, 64, 256
_HW = _H * _W
_R = (_O * _HW) // 128


def _period_kernel(labels_ref, emb_ref, w_ref, b_ref, out_ref, tiles_ref,
                   sems):
    feats = jax.lax.dot_general(
        w_ref[:], emb_ref[:], (((1,), (1,)), ((), ())),
        preferred_element_type=jnp.float32) + b_ref[:]

    r_i = jax.lax.broadcasted_iota(jnp.int32, (_R, _O), 0)
    c_i = jax.lax.broadcasted_iota(jnp.int32, (_R, _O), 1)
    cs = jax.lax.shift_right_logical(2 * r_i * 21400, 20)
    s0 = (cs == c_i).astype(jnp.float32)
    s1 = (cs + 1 == c_i).astype(jnp.float32)
    g0 = jnp.dot(s0, feats, preferred_element_type=jnp.float32)
    g1 = jnp.dot(s1, feats, preferred_element_type=jnp.float32)

    rl_i = jax.lax.broadcasted_iota(jnp.int32, (_R, 128), 0)
    l_i = jax.lax.broadcasted_iota(jnp.int32, (_R, 128), 1)
    csl = jax.lax.shift_right_logical(2 * rl_i * 21400, 20)
    in_first = 128 * rl_i + l_i < _HW * (csl + 1)
    for p in range(_NP):
        tiles_ref[p] = jnp.where(in_first, g0[:, p, None], g1[:, p, None])

    for b in range(_B):
        lab = labels_ref[b]
        pltpu.make_async_copy(
            tiles_ref.at[lab], out_ref.at[b], sems.at[b]).start()
    for b in range(_B):
        lab = labels_ref[b]
        pltpu.make_async_copy(
            tiles_ref.at[lab], out_ref.at[b], sems.at[b]).wait()


def kernel(period_labels, spatial_size, emb_table, fc_w, fc_b):
    fcb2d = fc_b.reshape(_O, 1)
    grid_spec = pltpu.PrefetchScalarGridSpec(
        num_scalar_prefetch=1,
        grid=(1,),
        in_specs=[
            pl.BlockSpec((_NP, _E), lambda i, s: (0, 0)),
            pl.BlockSpec((_O, _E), lambda i, s: (0, 0)),
            pl.BlockSpec((_O, 1), lambda i, s: (0, 0)),
        ],
        out_specs=pl.BlockSpec(memory_space=pl.ANY),
        scratch_shapes=[
            pltpu.VMEM((_NP, _R, 128), jnp.float32),
            pltpu.SemaphoreType.DMA((_B,)),
        ],
    )
    out = pl.pallas_call(
        _period_kernel,
        grid_spec=grid_spec,
        out_shape=jax.ShapeDtypeStruct((_B, _R, 128), jnp.float32),
    )(period_labels.astype(jnp.int32), emb_table, fc_w, fcb2d)
    return out.reshape(_B, _O, _H, _W)


# speedup vs baseline: 2.8491x; 2.8491x over previous
import jax
import jax.numpy as jnp
from jax.experimental import pallas as pl
from jax.experimental.pallas import tpu as pltpu

_B, _H, _W = 32, 56, 56
_NP, _E, _O = 4, 64, 256
_HW = _H * _W


def _period_kernel(labels_ref, emb_ref, w_ref, b_ref, out_ref, tiles_ref,
                   sems):
    feats = jax.lax.dot_general(
        emb_ref[:], w_ref[:], (((1,), (1,)), ((), ())),
        preferred_element_type=jnp.float32) + b_ref[:]
    tiles_ref[:] = jnp.broadcast_to(feats[:, :, None], (_NP, _O, _HW))
    for b in range(_B):
        lab = labels_ref[b]
        pltpu.make_async_copy(
            tiles_ref.at[lab], out_ref.at[b], sems.at[b]).start()
    for b in range(_B):
        lab = labels_ref[b]
        pltpu.make_async_copy(
            tiles_ref.at[lab], out_ref.at[b], sems.at[b]).wait()


def kernel(period_labels, spatial_size, emb_table, fc_w, fc_b):
    fcb2d = fc_b.reshape(1, _O)
    grid_spec = pltpu.PrefetchScalarGridSpec(
        num_scalar_prefetch=1,
        grid=(1,),
        in_specs=[
            pl.BlockSpec((_NP, _E), lambda i, s: (0, 0)),
            pl.BlockSpec((_O, _E), lambda i, s: (0, 0)),
            pl.BlockSpec((1, _O), lambda i, s: (0, 0)),
        ],
        out_specs=pl.BlockSpec(memory_space=pl.ANY),
        scratch_shapes=[
            pltpu.VMEM((_NP, _O, _HW), jnp.float32),
            pltpu.SemaphoreType.DMA((_B,)),
        ],
    )
    out = pl.pallas_call(
        _period_kernel,
        grid_spec=grid_spec,
        out_shape=jax.ShapeDtypeStruct((_B, _O, _HW), jnp.float32),
    )(period_labels.astype(jnp.int32), emb_table, fc_w, fcb2d)
    return out.reshape(_B, _O, _H, _W)


# speedup vs baseline: 10.4347x; 3.6624x over previous
import jax
import jax.numpy as jnp
from jax.experimental import pallas as pl
from jax.experimental.pallas import tpu as pltpu

_B, _H, _W = 32, 56, 56
_NP, _E, _O = 4, 64, 256
_HW = _H * _W


def _period_kernel(labels_ref, emb_ref, w_ref, b_ref, out_ref, tiles_ref,
                   sems):
    feats = jax.lax.dot_general(
        emb_ref[:], w_ref[:], (((1,), (1,)), ((), ())),
        preferred_element_type=jnp.float32) + b_ref[:]
    tiles_ref[:] = jnp.broadcast_to(feats[:, None, :], (_NP, _HW, _O))
    for b in range(_B):
        lab = labels_ref[b]
        pltpu.make_async_copy(
            tiles_ref.at[lab], out_ref.at[b], sems.at[b]).start()
    for b in range(_B):
        lab = labels_ref[b]
        pltpu.make_async_copy(
            tiles_ref.at[lab], out_ref.at[b], sems.at[b]).wait()


def kernel(period_labels, spatial_size, emb_table, fc_w, fc_b):
    fcb2d = fc_b.reshape(1, _O)
    grid_spec = pltpu.PrefetchScalarGridSpec(
        num_scalar_prefetch=1,
        grid=(1,),
        in_specs=[
            pl.BlockSpec((_NP, _E), lambda i, s: (0, 0)),
            pl.BlockSpec((_O, _E), lambda i, s: (0, 0)),
            pl.BlockSpec((1, _O), lambda i, s: (0, 0)),
        ],
        out_specs=pl.BlockSpec(memory_space=pl.ANY),
        scratch_shapes=[
            pltpu.VMEM((_NP, _HW, _O), jnp.float32),
            pltpu.SemaphoreType.DMA((_B,)),
        ],
    )
    out = pl.pallas_call(
        _period_kernel,
        grid_spec=grid_spec,
        out_shape=jax.ShapeDtypeStruct((_B, _HW, _O), jnp.float32),
    )(period_labels.astype(jnp.int32), emb_table, fc_w, fcb2d)
    out = out.reshape(_B, _H, _W, _O)
    return jnp.transpose(out, (0, 3, 1, 2))


# speedup vs baseline: 10.7311x; 1.0284x over previous
import jax
import jax.numpy as jnp
from jax.experimental import pallas as pl
from jax.experimental.pallas import tpu as pltpu

_B, _H, _W = 32, 56, 56
_NP, _E, _O = 4, 64, 256
_HW = _H * _W


def _period_kernel(labels_ref, emb_ref, w_ref, b_ref, out_ref, tiles_ref,
                   sems):
    feats = jax.lax.dot_general(
        emb_ref[:], w_ref[:], (((1,), (1,)), ((), ())),
        preferred_element_type=jnp.float32) + b_ref[:]
    for p in range(_NP):
        tiles_ref[p] = jnp.broadcast_to(feats[p, None, :], (_HW, _O))
        for b in range(_B):
            @pl.when(labels_ref[b] == p)
            def _start():
                pltpu.make_async_copy(
                    tiles_ref.at[p], out_ref.at[b], sems.at[b]).start()
    for b in range(_B):
        lab = labels_ref[b]
        pltpu.make_async_copy(
            tiles_ref.at[lab], out_ref.at[b], sems.at[b]).wait()


def kernel(period_labels, spatial_size, emb_table, fc_w, fc_b):
    fcb2d = fc_b.reshape(1, _O)
    grid_spec = pltpu.PrefetchScalarGridSpec(
        num_scalar_prefetch=1,
        grid=(1,),
        in_specs=[
            pl.BlockSpec((_NP, _E), lambda i, s: (0, 0)),
            pl.BlockSpec((_O, _E), lambda i, s: (0, 0)),
            pl.BlockSpec((1, _O), lambda i, s: (0, 0)),
        ],
        out_specs=pl.BlockSpec(memory_space=pl.ANY),
        scratch_shapes=[
            pltpu.VMEM((_NP, _HW, _O), jnp.float32),
            pltpu.SemaphoreType.DMA((_B,)),
        ],
    )
    out = pl.pallas_call(
        _period_kernel,
        grid_spec=grid_spec,
        out_shape=jax.ShapeDtypeStruct((_B, _HW, _O), jnp.float32),
    )(period_labels.astype(jnp.int32), emb_table, fc_w, fcb2d)
    out = out.reshape(_B, _H, _W, _O)
    return jnp.transpose(out, (0, 3, 1, 2))
